# baseline (speedup 1.0000x reference)
import jax
import jax.numpy as jnp
from jax.experimental import pallas as pl
from jax.experimental.pallas import tpu as pltpu

BN = 1024


def _body(x_ref, e_ref, dist_ref, ind_ref, q_ref):
    xb = x_ref[...]
    e = e_ref[...]
    d = jax.lax.dot_general(xb, e, (((1,), (1,)), ((), ())),
                            preferred_element_type=jnp.float32)
    dist_ref[...] = d
    idx = jnp.argmax(d, axis=-1).astype(jnp.int32)
    ind_ref[0, 0, :] = idx
    oh = (jax.lax.broadcasted_iota(jnp.int32, d.shape, 1) == idx[:, None]
          ).astype(jnp.bfloat16)
    q_ref[...] = jax.lax.dot_general(oh, e.astype(jnp.bfloat16),
                                     (((1,), (0,)), ((), ())),
                                     preferred_element_type=jnp.float32)


def kernel(x, embed):
    x = x.astype(jnp.float32)
    b, n, d = x.shape
    h, c, _ = embed.shape
    N = b * n
    xf = x.reshape(N, d)
    ef = embed.reshape(c, d)
    grid = (N // BN,)
    dist, ind3, quant = pl.pallas_call(
        _body,
        grid=grid,
        in_specs=[
            pl.BlockSpec((BN, d), lambda i: (i, 0)),
            pl.BlockSpec((c, d), lambda i: (0, 0)),
        ],
        out_specs=[
            pl.BlockSpec((BN, c), lambda i: (i, 0)),
            pl.BlockSpec((1, 1, BN), lambda i: (i, 0, 0)),
            pl.BlockSpec((BN, d), lambda i: (i, 0)),
        ],
        out_shape=[
            jax.ShapeDtypeStruct((N, c), jnp.float32),
            jax.ShapeDtypeStruct((N // BN, 1, BN), jnp.int32),
            jax.ShapeDtypeStruct((N, d), jnp.float32),
        ],
    )(xf, ef)
    quantize = quant.reshape(b, n, d)
    embed_ind = ind3.reshape(b, n)
    dist_out = dist.reshape(h, b, n, c)
    return quantize, embed_ind, dist_out


# speedup vs baseline: 1.0628x; 1.0628x over previous
import jax
import jax.numpy as jnp
from jax.experimental import pallas as pl
from jax.experimental.pallas import tpu as pltpu

BN = 4096


def _body(x_ref, e_ref, dist_ref, ind_ref, q_ref):
    xb = x_ref[...]
    e = e_ref[...]
    d = jax.lax.dot_general(xb, e, (((1,), (1,)), ((), ())),
                            preferred_element_type=jnp.float32)
    dist_ref[...] = d
    idx = jnp.argmax(d, axis=-1).astype(jnp.int32)
    ind_ref[0, 0, :] = idx
    oh = (jax.lax.broadcasted_iota(jnp.int32, d.shape, 1) == idx[:, None]
          ).astype(jnp.bfloat16)
    q_ref[...] = jax.lax.dot_general(oh, e.astype(jnp.bfloat16),
                                     (((1,), (0,)), ((), ())),
                                     preferred_element_type=jnp.float32)


def kernel(x, embed):
    x = x.astype(jnp.float32)
    b, n, d = x.shape
    h, c, _ = embed.shape
    N = b * n
    xf = x.reshape(N, d)
    ef = embed.reshape(c, d)
    grid = (N // BN,)
    dist, ind3, quant = pl.pallas_call(
        _body,
        grid=grid,
        in_specs=[
            pl.BlockSpec((BN, d), lambda i: (i, 0)),
            pl.BlockSpec((c, d), lambda i: (0, 0)),
        ],
        out_specs=[
            pl.BlockSpec((BN, c), lambda i: (i, 0)),
            pl.BlockSpec((1, 1, BN), lambda i: (i, 0, 0)),
            pl.BlockSpec((BN, d), lambda i: (i, 0)),
        ],
        out_shape=[
            jax.ShapeDtypeStruct((N, c), jnp.float32),
            jax.ShapeDtypeStruct((N // BN, 1, BN), jnp.int32),
            jax.ShapeDtypeStruct((N, d), jnp.float32),
        ],
    )(xf, ef)
    quantize = quant.reshape(b, n, d)
    embed_ind = ind3.reshape(b, n)
    dist_out = dist.reshape(h, b, n, c)
    return quantize, embed_ind, dist_out


# speedup vs baseline: 1.1024x; 1.0373x over previous
import jax
import jax.numpy as jnp
from jax.experimental import pallas as pl
from jax.experimental.pallas import tpu as pltpu

BN = 2048
BQ = 2048


def _dist_body(x_ref, e_ref, dist_ref, ind_ref):
    xb = x_ref[...]
    e = e_ref[...]
    d = jax.lax.dot_general(xb, e, (((1,), (1,)), ((), ())),
                            preferred_element_type=jnp.float32)
    dist_ref[...] = d
    ind_ref[0, 0, :] = jnp.argmax(d, axis=-1).astype(jnp.int32)


def _quant_body(ind_ref, e_ref, q_ref):
    idx = ind_ref[0, 0, :]
    e = e_ref[...]
    oh = (jax.lax.broadcasted_iota(jnp.int32, (idx.shape[0], e.shape[0]), 1)
          == idx[:, None]).astype(jnp.bfloat16)
    q_ref[...] = jax.lax.dot_general(oh, e.astype(jnp.bfloat16),
                                     (((1,), (0,)), ((), ())),
                                     preferred_element_type=jnp.float32)


def kernel(x, embed):
    x = x.astype(jnp.float32)
    b, n, d = x.shape
    h, c, _ = embed.shape
    N = b * n
    xf = x.reshape(N, d)
    ef = embed.reshape(c, d)
    dist, ind3 = pl.pallas_call(
        _dist_body,
        grid=(N // BN,),
        in_specs=[
            pl.BlockSpec((BN, d), lambda i: (i, 0)),
            pl.BlockSpec((c, d), lambda i: (0, 0)),
        ],
        out_specs=[
            pl.BlockSpec((BN, c), lambda i: (i, 0)),
            pl.BlockSpec((1, 1, BN), lambda i: (i, 0, 0)),
        ],
        out_shape=[
            jax.ShapeDtypeStruct((N, c), jnp.float32),
            jax.ShapeDtypeStruct((N // BN, 1, BN), jnp.int32),
        ],
    )(xf, ef)
    ind_q = ind3.reshape(N // BQ, 1, BQ)
    quant = pl.pallas_call(
        _quant_body,
        grid=(N // BQ,),
        in_specs=[
            pl.BlockSpec((1, 1, BQ), lambda i: (i, 0, 0)),
            pl.BlockSpec((c, d), lambda i: (0, 0)),
        ],
        out_specs=pl.BlockSpec((BQ, d), lambda i: (i, 0)),
        out_shape=jax.ShapeDtypeStruct((N, d), jnp.float32),
    )(ind_q, ef)
    quantize = quant.reshape(b, n, d)
    embed_ind = ind3.reshape(b, n)
    dist_out = dist.reshape(h, b, n, c)
    return quantize, embed_ind, dist_out


# speedup vs baseline: 1.1557x; 1.0484x over previous
import jax
import jax.numpy as jnp
from jax.experimental import pallas as pl
from jax.experimental.pallas import tpu as pltpu

BN = 2048
IW = 128


def _body(x_ref, e_ref, dist_ref, ind_ref, q_ref):
    xb = x_ref[...]
    e = e_ref[...]
    c = e.shape[0]
    d = jax.lax.dot_general(xb, e, (((1,), (1,)), ((), ())),
                            preferred_element_type=jnp.float32)
    dist_ref[...] = d
    m = jnp.max(d, axis=-1, keepdims=True)
    iota = jax.lax.broadcasted_iota(jnp.int32, d.shape, 1).astype(jnp.float32)
    w = jnp.where(d == m, iota, float(c))
    idx = jnp.min(w, axis=-1, keepdims=True)
    ind_ref[...] = jnp.broadcast_to(idx.astype(jnp.int32), (xb.shape[0], IW))
    oh = (iota == idx).astype(jnp.bfloat16)
    q_ref[...] = jax.lax.dot_general(oh, e.astype(jnp.bfloat16),
                                     (((1,), (0,)), ((), ())),
                                     preferred_element_type=jnp.float32)


def kernel(x, embed):
    x = x.astype(jnp.float32)
    b, n, d = x.shape
    h, c, _ = embed.shape
    N = b * n
    xf = x.reshape(N, d)
    ef = embed.reshape(c, d)
    dist, ind_wide, quant = pl.pallas_call(
        _body,
        grid=(N // BN,),
        in_specs=[
            pl.BlockSpec((BN, d), lambda i: (i, 0)),
            pl.BlockSpec((c, d), lambda i: (0, 0)),
        ],
        out_specs=[
            pl.BlockSpec((BN, c), lambda i: (i, 0)),
            pl.BlockSpec((BN, IW), lambda i: (i, 0)),
            pl.BlockSpec((BN, d), lambda i: (i, 0)),
        ],
        out_shape=[
            jax.ShapeDtypeStruct((N, c), jnp.float32),
            jax.ShapeDtypeStruct((N, IW), jnp.int32),
            jax.ShapeDtypeStruct((N, d), jnp.float32),
        ],
    )(xf, ef)
    quantize = quant.reshape(b, n, d)
    embed_ind = ind_wide[:, 0].reshape(b, n)
    dist_out = dist.reshape(h, b, n, c)
    return quantize, embed_ind, dist_out


# speedup vs baseline: 1.1949x; 1.0340x over previous
import jax
import jax.numpy as jnp
from jax.experimental import pallas as pl
from jax.experimental.pallas import tpu as pltpu

BN = 2048
IW = 128


def _body(x_ref, e_ref, dist_ref, ind_ref, q_ref):
    xb = x_ref[...]
    e = e_ref[...]
    c = e.shape[0]
    d = jax.lax.dot_general(xb, e, (((1,), (1,)), ((), ())),
                            preferred_element_type=jnp.float32)
    dist_ref[...] = d
    m = jnp.max(d, axis=-1, keepdims=True)
    iota = jax.lax.broadcasted_iota(jnp.int32, d.shape, 1).astype(jnp.float32)
    w = jnp.where(d == m, iota, float(c))
    idx = jnp.min(w, axis=-1, keepdims=True)
    ind_ref[0, 0, :] = jnp.transpose(idx.astype(jnp.int32), (1, 0))[0]
    oh = (iota == idx).astype(jnp.bfloat16)
    q_ref[...] = jax.lax.dot_general(oh, e.astype(jnp.bfloat16),
                                     (((1,), (0,)), ((), ())),
                                     preferred_element_type=jnp.float32)


def kernel(x, embed):
    x = x.astype(jnp.float32)
    b, n, d = x.shape
    h, c, _ = embed.shape
    N = b * n
    xf = x.reshape(N, d)
    ef = embed.reshape(c, d)
    dist, ind_wide, quant = pl.pallas_call(
        _body,
        grid=(N // BN,),
        in_specs=[
            pl.BlockSpec((BN, d), lambda i: (i, 0)),
            pl.BlockSpec((c, d), lambda i: (0, 0)),
        ],
        out_specs=[
            pl.BlockSpec((BN, c), lambda i: (i, 0)),
            pl.BlockSpec((1, 1, BN), lambda i: (i, 0, 0)),
            pl.BlockSpec((BN, d), lambda i: (i, 0)),
        ],
        out_shape=[
            jax.ShapeDtypeStruct((N, c), jnp.float32),
            jax.ShapeDtypeStruct((N // BN, 1, BN), jnp.int32),
            jax.ShapeDtypeStruct((N, d), jnp.float32),
        ],
    )(xf, ef)
    quantize = quant.reshape(b, n, d)
    embed_ind = ind_wide.reshape(b, n)
    dist_out = dist.reshape(h, b, n, c)
    return quantize, embed_ind, dist_out
